# baseline (speedup 1.0000x reference)
import functools

import jax
import jax.numpy as jnp
from jax import lax
from jax.experimental import pallas as pl
from jax.experimental.pallas import tpu as pltpu
from jax.experimental.pallas import tpu_sc as plsc

N = 10000
D = 128
H = 128
E = 320000

NC = 2
NS = 16
F = D // NC
EPT = 20480
EP = NS * EPT
SG = 5
EPS = EPT // SG
C = 64
CPS = EPS // C
GB = 2
SB = 2
UNR = 2
ITERS = CPS // UNR
RPT = 624
TAIL = N - NS * RPT


def _spmm_sc(ht, src, dst, val, zeros):
    mesh = plsc.VectorSubcoreMesh(
        core_axis_name="c", subcore_axis_name="s", num_cores=NC)

    @functools.partial(
        pl.kernel,
        out_type=jax.ShapeDtypeStruct((NC, N, F), jnp.float32),
        mesh=mesh,
        compiler_params=pltpu.CompilerParams(use_tc_tiling_on_sc=False),
        scratch_types=[
            pltpu.VMEM_SHARED((N, F), jnp.float32),
            pltpu.VMEM((EPS,), jnp.int32),
            pltpu.VMEM((EPS,), jnp.int32),
            pltpu.VMEM((2, CPS, C), jnp.int32),
            pltpu.VMEM((EPS,), jnp.float32),
            pltpu.VMEM((EPS,), jnp.float32),
            pltpu.VMEM((GB, C, F), jnp.float32),
            pltpu.VMEM((SB, C, F), jnp.float32),
            pltpu.SemaphoreType.DMA((GB,)),
            pltpu.SemaphoreType.DMA((SB,)),
            pltpu.SemaphoreType.DMA,
        ],
    )
    def k(h_hbm, src_hbm, dst_hbm, val_hbm, zeros_hbm, out_hbm,
          acc, src_v0, src_v1, dst_v, val_v0, val_v1,
          rows_g, rows_s, gsem, ssem, psem):
        c = lax.axis_index("c")
        s = lax.axis_index("s")
        src_vs = (src_v0, src_v1)
        val_vs = (val_v0, val_v1)
        pltpu.sync_copy(zeros_hbm.at[pl.ds(s * RPT, RPT)],
                        acc.at[pl.ds(s * RPT, RPT)])

        @pl.when(s == NS - 1)
        def _():
            pltpu.sync_copy(zeros_hbm.at[pl.ds(NS * RPT, TAIL)],
                            acc.at[pl.ds(NS * RPT, TAIL)])

        coff = jnp.full((16,), c * N, jnp.int32)

        def add_off(sv):
            def step(i, _):
                sl = pl.ds(i * 16, 16)
                sv[sl] = sv[sl] + coff
                return 0
            lax.fori_loop(0, EPS // 16, step, 0)

        pltpu.sync_copy(src_hbm.at[s * SG], src_v0)
        pltpu.sync_copy(dst_hbm.at[s * SG], dst_v.at[0])
        pltpu.sync_copy(val_hbm.at[s * SG], val_v0)
        add_off(src_v0)
        plsc.subcore_barrier()

        def gather(tb, j, gb):
            return pltpu.async_copy(
                h_hbm.at[src_vs[tb].at[pl.ds(j * C, C)]], rows_g.at[gb],
                gsem.at[gb])

        def scatter_desc(tb, j, b):
            return pltpu.make_async_copy(
                rows_s.at[b], acc.at[dst_v.at[tb, j]], ssem.at[b])

        for b in range(GB):
            gather(0, b, b)

        for t in range(SG):
            tb, ntb = t % 2, (t + 1) % 2
            if t + 1 < SG:
                pltpu.async_copy(src_hbm.at[s * SG + t + 1], src_vs[ntb],
                                 psem)
                pltpu.async_copy(dst_hbm.at[s * SG + t + 1], dst_v.at[ntb],
                                 psem)
                pltpu.async_copy(val_hbm.at[s * SG + t + 1], val_vs[ntb],
                                 psem)

            def body(jj, _, t=t, tb=tb, ntb=ntb):
                if t + 1 < SG:
                    @pl.when(jj == ITERS - 1)
                    def _():
                        pltpu.make_async_copy(
                            src_hbm.at[s * SG + t + 1], src_vs[ntb],
                            psem).wait()
                        pltpu.make_async_copy(
                            dst_hbm.at[s * SG + t + 1], dst_v.at[ntb],
                            psem).wait()
                        pltpu.make_async_copy(
                            val_hbm.at[s * SG + t + 1], val_vs[ntb],
                            psem).wait()
                        add_off(src_vs[ntb])
                for b in range(UNR):
                    j = jj * UNR + b
                    gb = b % GB
                    pltpu.make_async_copy(
                        h_hbm.at[src_vs[tb].at[pl.ds(j * C, C)]],
                        rows_g.at[gb], gsem.at[gb]).wait()

                    def drain():
                        scatter_desc(tb, j, b).wait()
                    if t == 0:
                        pl.when(jj >= 1)(drain)
                    else:
                        drain()

                    for g in range(C // 16):
                        vg = val_vs[tb][pl.ds(j * C + g * 16, 16)]
                        for i in range(16):
                            vb = jnp.full((16,), vg[i], jnp.float32)
                            e = g * 16 + i
                            for kk in range(F // 16):
                                sl = pl.ds(kk * 16, 16)
                                rows_s[b, e, sl] = rows_g[gb, e, sl] * vb

                    pltpu.async_copy(rows_s.at[b], acc.at[dst_v.at[tb, j]],
                                     ssem.at[b], add=True)

                    @pl.when(jj < ITERS - 1)
                    def _():
                        gather(tb, j + GB, gb)
                    if t + 1 < SG:
                        @pl.when(jj == ITERS - 1)
                        def _():
                            gather(ntb, b, gb)
                return 0

            lax.fori_loop(0, ITERS, body, 0)

        for b in range(SB):
            scatter_desc((SG - 1) % 2, CPS - SB + b, b).wait()
        plsc.subcore_barrier()

        pltpu.sync_copy(acc.at[pl.ds(s * RPT, RPT)],
                        out_hbm.at[c, pl.ds(s * RPT, RPT)])

        @pl.when(s == NS - 1)
        def _():
            pltpu.sync_copy(acc.at[pl.ds(NS * RPT, TAIL)],
                            out_hbm.at[c, pl.ds(NS * RPT, TAIL)])

    return k(ht, src, dst, val, zeros)


BR = 400


def _mid_layer_tc(q, Wa, Wb, b):
    def body(q_ref, wa_ref, wb_ref, b_ref, o_ref):
        h = jnp.maximum(
            jnp.dot(q_ref[0], wa_ref[...], preferred_element_type=jnp.float32)
            + jnp.dot(q_ref[1], wb_ref[...], preferred_element_type=jnp.float32)
            + b_ref[...], 0.0)
        o_ref[0] = h[:, :F]
        o_ref[1] = h[:, F:]

    return pl.pallas_call(
        body,
        grid=(N // BR,),
        in_specs=[
            pl.BlockSpec((NC, BR, F), lambda i: (0, i, 0)),
            pl.BlockSpec((F, H), lambda i: (0, 0)),
            pl.BlockSpec((F, H), lambda i: (0, 0)),
            pl.BlockSpec((1, H), lambda i: (0, 0)),
        ],
        out_specs=pl.BlockSpec((NC, BR, F), lambda i: (0, i, 0)),
        out_shape=jax.ShapeDtypeStruct((NC, N, F), jnp.float32),
    )(q, Wa, Wb, b.reshape(1, H))


def _final_layer_tc(r, W1a, W1b, b1, W_out, b_out):
    def body(r_ref, wa_ref, wb_ref, b1_ref, wo_ref, bo_ref, o_ref):
        t = jnp.maximum(
            jnp.dot(r_ref[0], wa_ref[...], preferred_element_type=jnp.float32)
            + jnp.dot(r_ref[1], wb_ref[...], preferred_element_type=jnp.float32)
            + b1_ref[...], 0.0)
        o_ref[...] = jnp.dot(t, wo_ref[...],
                             preferred_element_type=jnp.float32) + bo_ref[...]

    return pl.pallas_call(
        body,
        grid=(N // BR,),
        in_specs=[
            pl.BlockSpec((NC, BR, F), lambda i: (0, i, 0)),
            pl.BlockSpec((F, H), lambda i: (0, 0)),
            pl.BlockSpec((F, H), lambda i: (0, 0)),
            pl.BlockSpec((1, H), lambda i: (0, 0)),
            pl.BlockSpec((H, 1), lambda i: (0, 0)),
            pl.BlockSpec((1, 1), lambda i: (0, 0)),
        ],
        out_specs=pl.BlockSpec((BR, 1), lambda i: (i, 0)),
        out_shape=jax.ShapeDtypeStruct((N, 1), jnp.float32),
    )(r, W1a, W1b, b1.reshape(1, H), W_out, b_out.reshape(1, 1))


def kernel(x, adj_indices, adj_values, W0, b0, W1, b1, W_out, b_out):
    pad = EP - E
    dst32 = jnp.concatenate(
        [adj_indices[0].astype(jnp.int32),
         jnp.arange(pad, dtype=jnp.int32) % N])
    src32 = jnp.concatenate(
        [adj_indices[1].astype(jnp.int32),
         jnp.arange(pad, dtype=jnp.int32) % N])
    valp = jnp.concatenate(
        [adj_values.astype(jnp.float32), jnp.zeros((pad,), jnp.float32)])
    src = src32.reshape(NS * SG, EPS)
    dst = dst32.reshape(NS * SG, CPS, C)
    val = valp.reshape(NS * SG, EPS)
    zeros = jnp.zeros((N, F), jnp.float32)

    xt = jnp.concatenate([x[:, :F], x[:, F:]], axis=0)
    q = _spmm_sc(xt, src, dst, val, zeros)
    h1 = _mid_layer_tc(q, W0[:F], W0[F:], b0)
    r = _spmm_sc(h1.reshape(NC * N, F), src, dst, val, zeros)
    out = _final_layer_tc(r, W1[:F], W1[F:], b1, W_out, b_out)
    return out[:, 0]


# speedup vs baseline: 1.1671x; 1.1671x over previous
import functools

import jax
import jax.numpy as jnp
from jax import lax
from jax.experimental import pallas as pl
from jax.experimental.pallas import tpu as pltpu
from jax.experimental.pallas import tpu_sc as plsc

N = 10000
D = 128
H = 128
E = 320000

NC = 2
NS = 16
F = D // NC
EPT = 20480
EP = NS * EPT
SG = 5
EPS = EPT // SG
C = 128
CPS = EPS // C
GB = 2
SB = 2
UNR = 2
ITERS = CPS // UNR
RPT = 624
TAIL = N - NS * RPT


def _spmm_sc(ht, src, dst, val, zeros):
    mesh = plsc.VectorSubcoreMesh(
        core_axis_name="c", subcore_axis_name="s", num_cores=NC)

    @functools.partial(
        pl.kernel,
        out_type=jax.ShapeDtypeStruct((NC, N, F), jnp.float32),
        mesh=mesh,
        compiler_params=pltpu.CompilerParams(use_tc_tiling_on_sc=False),
        scratch_types=[
            pltpu.VMEM_SHARED((N, F), jnp.float32),
            pltpu.VMEM((EPS,), jnp.int32),
            pltpu.VMEM((EPS,), jnp.int32),
            pltpu.VMEM((2, CPS, C), jnp.int32),
            pltpu.VMEM((EPS,), jnp.float32),
            pltpu.VMEM((EPS,), jnp.float32),
            pltpu.VMEM((GB, C, F), jnp.float32),
            pltpu.VMEM((SB, C, F), jnp.float32),
            pltpu.SemaphoreType.DMA((GB,)),
            pltpu.SemaphoreType.DMA((SB,)),
            pltpu.SemaphoreType.DMA,
        ],
    )
    def k(h_hbm, src_hbm, dst_hbm, val_hbm, zeros_hbm, out_hbm,
          acc, src_v0, src_v1, dst_v, val_v0, val_v1,
          rows_g, rows_s, gsem, ssem, psem):
        c = lax.axis_index("c")
        s = lax.axis_index("s")
        src_vs = (src_v0, src_v1)
        val_vs = (val_v0, val_v1)
        pltpu.sync_copy(zeros_hbm.at[pl.ds(s * RPT, RPT)],
                        acc.at[pl.ds(s * RPT, RPT)])

        @pl.when(s == NS - 1)
        def _():
            pltpu.sync_copy(zeros_hbm.at[pl.ds(NS * RPT, TAIL)],
                            acc.at[pl.ds(NS * RPT, TAIL)])

        coff = jnp.full((16,), c * N, jnp.int32)

        def add_off(sv):
            def step(i, _):
                sl = pl.ds(i * 16, 16)
                sv[sl] = sv[sl] + coff
                return 0
            lax.fori_loop(0, EPS // 16, step, 0)

        pltpu.sync_copy(src_hbm.at[s * SG], src_v0)
        pltpu.sync_copy(dst_hbm.at[s * SG], dst_v.at[0])
        pltpu.sync_copy(val_hbm.at[s * SG], val_v0)
        add_off(src_v0)
        plsc.subcore_barrier()

        def gather(tb, j, gb):
            return pltpu.async_copy(
                h_hbm.at[src_vs[tb].at[pl.ds(j * C, C)]], rows_g.at[gb],
                gsem.at[gb])

        def scatter_desc(tb, j, b):
            return pltpu.make_async_copy(
                rows_s.at[b], acc.at[dst_v.at[tb, j]], ssem.at[b])

        for b in range(GB):
            gather(0, b, b)

        for t in range(SG):
            tb, ntb = t % 2, (t + 1) % 2
            if t + 1 < SG:
                pltpu.async_copy(src_hbm.at[s * SG + t + 1], src_vs[ntb],
                                 psem)
                pltpu.async_copy(dst_hbm.at[s * SG + t + 1], dst_v.at[ntb],
                                 psem)
                pltpu.async_copy(val_hbm.at[s * SG + t + 1], val_vs[ntb],
                                 psem)

            def body(jj, _, t=t, tb=tb, ntb=ntb):
                if t + 1 < SG:
                    @pl.when(jj == ITERS - 1)
                    def _():
                        pltpu.make_async_copy(
                            src_hbm.at[s * SG + t + 1], src_vs[ntb],
                            psem).wait()
                        pltpu.make_async_copy(
                            dst_hbm.at[s * SG + t + 1], dst_v.at[ntb],
                            psem).wait()
                        pltpu.make_async_copy(
                            val_hbm.at[s * SG + t + 1], val_vs[ntb],
                            psem).wait()
                        add_off(src_vs[ntb])
                for b in range(UNR):
                    j = jj * UNR + b
                    gb = b % GB
                    pltpu.make_async_copy(
                        h_hbm.at[src_vs[tb].at[pl.ds(j * C, C)]],
                        rows_g.at[gb], gsem.at[gb]).wait()

                    def drain():
                        scatter_desc(tb, j, b).wait()
                    if t == 0:
                        pl.when(jj >= 1)(drain)
                    else:
                        drain()

                    for g in range(C // 16):
                        vg = val_vs[tb][pl.ds(j * C + g * 16, 16)]
                        for i in range(16):
                            vb = jnp.full((16,), vg[i], jnp.float32)
                            e = g * 16 + i
                            for kk in range(F // 16):
                                sl = pl.ds(kk * 16, 16)
                                rows_s[b, e, sl] = rows_g[gb, e, sl] * vb

                    pltpu.async_copy(rows_s.at[b], acc.at[dst_v.at[tb, j]],
                                     ssem.at[b], add=True)

                    @pl.when(jj < ITERS - 1)
                    def _():
                        gather(tb, j + GB, gb)
                    if t + 1 < SG:
                        @pl.when(jj == ITERS - 1)
                        def _():
                            gather(ntb, b, gb)
                return 0

            lax.fori_loop(0, ITERS, body, 0)

        for b in range(SB):
            scatter_desc((SG - 1) % 2, CPS - SB + b, b).wait()
        plsc.subcore_barrier()

        pltpu.sync_copy(acc.at[pl.ds(s * RPT, RPT)],
                        out_hbm.at[c, pl.ds(s * RPT, RPT)])

        @pl.when(s == NS - 1)
        def _():
            pltpu.sync_copy(acc.at[pl.ds(NS * RPT, TAIL)],
                            out_hbm.at[c, pl.ds(NS * RPT, TAIL)])

    return k(ht, src, dst, val, zeros)


BR = 400


def _mid_layer_tc(q, Wa, Wb, b):
    def body(q_ref, wa_ref, wb_ref, b_ref, o_ref):
        h = jnp.maximum(
            jnp.dot(q_ref[0], wa_ref[...], preferred_element_type=jnp.float32)
            + jnp.dot(q_ref[1], wb_ref[...], preferred_element_type=jnp.float32)
            + b_ref[...], 0.0)
        o_ref[0] = h[:, :F]
        o_ref[1] = h[:, F:]

    return pl.pallas_call(
        body,
        grid=(N // BR,),
        in_specs=[
            pl.BlockSpec((NC, BR, F), lambda i: (0, i, 0)),
            pl.BlockSpec((F, H), lambda i: (0, 0)),
            pl.BlockSpec((F, H), lambda i: (0, 0)),
            pl.BlockSpec((1, H), lambda i: (0, 0)),
        ],
        out_specs=pl.BlockSpec((NC, BR, F), lambda i: (0, i, 0)),
        out_shape=jax.ShapeDtypeStruct((NC, N, F), jnp.float32),
    )(q, Wa, Wb, b.reshape(1, H))


def _final_layer_tc(r, W1a, W1b, b1, W_out, b_out):
    def body(r_ref, wa_ref, wb_ref, b1_ref, wo_ref, bo_ref, o_ref):
        t = jnp.maximum(
            jnp.dot(r_ref[0], wa_ref[...], preferred_element_type=jnp.float32)
            + jnp.dot(r_ref[1], wb_ref[...], preferred_element_type=jnp.float32)
            + b1_ref[...], 0.0)
        o_ref[...] = jnp.dot(t, wo_ref[...],
                             preferred_element_type=jnp.float32) + bo_ref[...]

    return pl.pallas_call(
        body,
        grid=(N // BR,),
        in_specs=[
            pl.BlockSpec((NC, BR, F), lambda i: (0, i, 0)),
            pl.BlockSpec((F, H), lambda i: (0, 0)),
            pl.BlockSpec((F, H), lambda i: (0, 0)),
            pl.BlockSpec((1, H), lambda i: (0, 0)),
            pl.BlockSpec((H, 1), lambda i: (0, 0)),
            pl.BlockSpec((1, 1), lambda i: (0, 0)),
        ],
        out_specs=pl.BlockSpec((BR, 1), lambda i: (i, 0)),
        out_shape=jax.ShapeDtypeStruct((N, 1), jnp.float32),
    )(r, W1a, W1b, b1.reshape(1, H), W_out, b_out.reshape(1, 1))


def kernel(x, adj_indices, adj_values, W0, b0, W1, b1, W_out, b_out):
    pad = EP - E
    dst32 = jnp.concatenate(
        [adj_indices[0].astype(jnp.int32),
         jnp.arange(pad, dtype=jnp.int32) % N])
    src32 = jnp.concatenate(
        [adj_indices[1].astype(jnp.int32),
         jnp.arange(pad, dtype=jnp.int32) % N])
    valp = jnp.concatenate(
        [adj_values.astype(jnp.float32), jnp.zeros((pad,), jnp.float32)])
    src = src32.reshape(NS * SG, EPS)
    dst = dst32.reshape(NS * SG, CPS, C)
    val = valp.reshape(NS * SG, EPS)
    zeros = jnp.zeros((N, F), jnp.float32)

    xt = jnp.concatenate([x[:, :F], x[:, F:]], axis=0)
    q = _spmm_sc(xt, src, dst, val, zeros)
    h1 = _mid_layer_tc(q, W0[:F], W0[F:], b0)
    r = _spmm_sc(h1.reshape(NC * N, F), src, dst, val, zeros)
    out = _final_layer_tc(r, W1[:F], W1[F:], b1, W_out, b_out)
    return out[:, 0]


# speedup vs baseline: 1.2107x; 1.0374x over previous
import functools

import jax
import jax.numpy as jnp
from jax import lax
from jax.experimental import pallas as pl
from jax.experimental.pallas import tpu as pltpu
from jax.experimental.pallas import tpu_sc as plsc

N = 10000
D = 128
H = 128
E = 320000

NC = 2
NS = 16
F = D // NC
EPT = 20480
EP = NS * EPT
SG = 5
EPS = EPT // SG
C = 128
CPS = EPS // C
GB = 2
SB = 4
UNR = 4
ITERS = CPS // UNR
RPT = 624
TAIL = N - NS * RPT


def _spmm_sc(ht, src, dst, val, zeros):
    mesh = plsc.VectorSubcoreMesh(
        core_axis_name="c", subcore_axis_name="s", num_cores=NC)

    @functools.partial(
        pl.kernel,
        out_type=jax.ShapeDtypeStruct((NC, N, F), jnp.float32),
        mesh=mesh,
        compiler_params=pltpu.CompilerParams(use_tc_tiling_on_sc=False),
        scratch_types=[
            pltpu.VMEM_SHARED((N, F), jnp.float32),
            pltpu.VMEM((EPS,), jnp.int32),
            pltpu.VMEM((EPS,), jnp.int32),
            pltpu.VMEM((2, CPS, C), jnp.int32),
            pltpu.VMEM((EPS,), jnp.float32),
            pltpu.VMEM((EPS,), jnp.float32),
            pltpu.VMEM((GB, C, F), jnp.float32),
            pltpu.VMEM((SB, C, F), jnp.float32),
            pltpu.SemaphoreType.DMA((GB,)),
            pltpu.SemaphoreType.DMA((SB,)),
            pltpu.SemaphoreType.DMA,
        ],
    )
    def k(h_hbm, src_hbm, dst_hbm, val_hbm, zeros_hbm, out_hbm,
          acc, src_v0, src_v1, dst_v, val_v0, val_v1,
          rows_g, rows_s, gsem, ssem, psem):
        c = lax.axis_index("c")
        s = lax.axis_index("s")
        src_vs = (src_v0, src_v1)
        val_vs = (val_v0, val_v1)
        pltpu.sync_copy(zeros_hbm.at[pl.ds(s * RPT, RPT)],
                        acc.at[pl.ds(s * RPT, RPT)])

        @pl.when(s == NS - 1)
        def _():
            pltpu.sync_copy(zeros_hbm.at[pl.ds(NS * RPT, TAIL)],
                            acc.at[pl.ds(NS * RPT, TAIL)])

        coff = jnp.full((16,), c * N, jnp.int32)

        def add_off(sv):
            def step(i, _):
                sl = pl.ds(i * 16, 16)
                sv[sl] = sv[sl] + coff
                return 0
            lax.fori_loop(0, EPS // 16, step, 0)

        pltpu.sync_copy(src_hbm.at[s * SG], src_v0)
        pltpu.sync_copy(dst_hbm.at[s * SG], dst_v.at[0])
        pltpu.sync_copy(val_hbm.at[s * SG], val_v0)
        add_off(src_v0)
        plsc.subcore_barrier()

        def gather(tb, j, gb):
            return pltpu.async_copy(
                h_hbm.at[src_vs[tb].at[pl.ds(j * C, C)]], rows_g.at[gb],
                gsem.at[gb])

        def scatter_desc(tb, j, b):
            return pltpu.make_async_copy(
                rows_s.at[b], acc.at[dst_v.at[tb, j]], ssem.at[b])

        for b in range(GB):
            gather(0, b, b)

        for t in range(SG):
            tb, ntb = t % 2, (t + 1) % 2
            if t + 1 < SG:
                pltpu.async_copy(src_hbm.at[s * SG + t + 1], src_vs[ntb],
                                 psem)
                pltpu.async_copy(dst_hbm.at[s * SG + t + 1], dst_v.at[ntb],
                                 psem)
                pltpu.async_copy(val_hbm.at[s * SG + t + 1], val_vs[ntb],
                                 psem)

            def body(jj, _, t=t, tb=tb, ntb=ntb):
                if t + 1 < SG:
                    @pl.when(jj == ITERS - 1)
                    def _():
                        pltpu.make_async_copy(
                            src_hbm.at[s * SG + t + 1], src_vs[ntb],
                            psem).wait()
                        pltpu.make_async_copy(
                            dst_hbm.at[s * SG + t + 1], dst_v.at[ntb],
                            psem).wait()
                        pltpu.make_async_copy(
                            val_hbm.at[s * SG + t + 1], val_vs[ntb],
                            psem).wait()
                        add_off(src_vs[ntb])
                for b in range(UNR):
                    j = jj * UNR + b
                    gb = b % GB
                    pltpu.make_async_copy(
                        h_hbm.at[src_vs[tb].at[pl.ds(j * C, C)]],
                        rows_g.at[gb], gsem.at[gb]).wait()

                    def drain():
                        scatter_desc(tb, j, b).wait()
                    if t == 0:
                        pl.when(jj >= 1)(drain)
                    else:
                        drain()

                    def group(g, _, b=b, gb=gb, j=j):
                        vg = val_vs[tb][pl.ds(j * C + g * 16, 16)]
                        for i in range(16):
                            vb = jnp.full((16,), vg[i], jnp.float32)
                            e = g * 16 + i
                            for kk in range(F // 16):
                                sl = pl.ds(kk * 16, 16)
                                rows_s[b, e, sl] = rows_g[gb, e, sl] * vb
                        return 0

                    lax.fori_loop(0, C // 16, group, 0)

                    pltpu.async_copy(rows_s.at[b], acc.at[dst_v.at[tb, j]],
                                     ssem.at[b], add=True)

                    if b < UNR - GB:
                        gather(tb, j + GB, gb)
                    else:
                        @pl.when(jj < ITERS - 1)
                        def _():
                            gather(tb, j + GB, gb)
                        if t + 1 < SG:
                            @pl.when(jj == ITERS - 1)
                            def _():
                                gather(ntb, b - (UNR - GB), gb)
                return 0

            lax.fori_loop(0, ITERS, body, 0)

        for b in range(SB):
            scatter_desc((SG - 1) % 2, CPS - SB + b, b).wait()
        plsc.subcore_barrier()

        pltpu.sync_copy(acc.at[pl.ds(s * RPT, RPT)],
                        out_hbm.at[c, pl.ds(s * RPT, RPT)])

        @pl.when(s == NS - 1)
        def _():
            pltpu.sync_copy(acc.at[pl.ds(NS * RPT, TAIL)],
                            out_hbm.at[c, pl.ds(NS * RPT, TAIL)])

    return k(ht, src, dst, val, zeros)


BR = 400


def _mid_layer_tc(q, Wa, Wb, b):
    def body(q_ref, wa_ref, wb_ref, b_ref, o_ref):
        h = jnp.maximum(
            jnp.dot(q_ref[0], wa_ref[...], preferred_element_type=jnp.float32)
            + jnp.dot(q_ref[1], wb_ref[...], preferred_element_type=jnp.float32)
            + b_ref[...], 0.0)
        o_ref[0] = h[:, :F]
        o_ref[1] = h[:, F:]

    return pl.pallas_call(
        body,
        grid=(N // BR,),
        in_specs=[
            pl.BlockSpec((NC, BR, F), lambda i: (0, i, 0)),
            pl.BlockSpec((F, H), lambda i: (0, 0)),
            pl.BlockSpec((F, H), lambda i: (0, 0)),
            pl.BlockSpec((1, H), lambda i: (0, 0)),
        ],
        out_specs=pl.BlockSpec((NC, BR, F), lambda i: (0, i, 0)),
        out_shape=jax.ShapeDtypeStruct((NC, N, F), jnp.float32),
    )(q, Wa, Wb, b.reshape(1, H))


def _final_layer_tc(r, W1a, W1b, b1, W_out, b_out):
    def body(r_ref, wa_ref, wb_ref, b1_ref, wo_ref, bo_ref, o_ref):
        t = jnp.maximum(
            jnp.dot(r_ref[0], wa_ref[...], preferred_element_type=jnp.float32)
            + jnp.dot(r_ref[1], wb_ref[...], preferred_element_type=jnp.float32)
            + b1_ref[...], 0.0)
        o_ref[...] = jnp.dot(t, wo_ref[...],
                             preferred_element_type=jnp.float32) + bo_ref[...]

    return pl.pallas_call(
        body,
        grid=(N // BR,),
        in_specs=[
            pl.BlockSpec((NC, BR, F), lambda i: (0, i, 0)),
            pl.BlockSpec((F, H), lambda i: (0, 0)),
            pl.BlockSpec((F, H), lambda i: (0, 0)),
            pl.BlockSpec((1, H), lambda i: (0, 0)),
            pl.BlockSpec((H, 1), lambda i: (0, 0)),
            pl.BlockSpec((1, 1), lambda i: (0, 0)),
        ],
        out_specs=pl.BlockSpec((BR, 1), lambda i: (i, 0)),
        out_shape=jax.ShapeDtypeStruct((N, 1), jnp.float32),
    )(r, W1a, W1b, b1.reshape(1, H), W_out, b_out.reshape(1, 1))


def kernel(x, adj_indices, adj_values, W0, b0, W1, b1, W_out, b_out):
    pad = EP - E
    dst32 = jnp.concatenate(
        [adj_indices[0].astype(jnp.int32),
         jnp.arange(pad, dtype=jnp.int32) % N])
    src32 = jnp.concatenate(
        [adj_indices[1].astype(jnp.int32),
         jnp.arange(pad, dtype=jnp.int32) % N])
    valp = jnp.concatenate(
        [adj_values.astype(jnp.float32), jnp.zeros((pad,), jnp.float32)])
    src = src32.reshape(NS * SG, EPS)
    dst = dst32.reshape(NS * SG, CPS, C)
    val = valp.reshape(NS * SG, EPS)
    zeros = jnp.zeros((N, F), jnp.float32)

    xt = jnp.concatenate([x[:, :F], x[:, F:]], axis=0)
    q = _spmm_sc(xt, src, dst, val, zeros)
    h1 = _mid_layer_tc(q, W0[:F], W0[F:], b0)
    r = _spmm_sc(h1.reshape(NC * N, F), src, dst, val, zeros)
    out = _final_layer_tc(r, W1[:F], W1[F:], b1, W_out, b_out)
    return out[:, 0]
